# baseline (speedup 1.0000x reference)
import functools

import jax
import jax.numpy as jnp
from jax import lax
from jax.experimental import pallas as pl
from jax.experimental.pallas import tpu as pltpu
from jax.experimental.pallas import tpu_sc as plsc

B = 16384
F = 26
D = 64
L = 16
NC = 2
NS = 16
NW = NC * NS
BPW = B // NW
C = 32
CF = C * F
NCHUNK = BPW // C


def _sc_body(idx_hbm, table_hbm, out_hbm, idx_v, rows_v, out_v, sem):
    wid = lax.axis_index("s") * NC + lax.axis_index("c")
    row0 = wid * BPW

    def chunk(t, carry):
        ibase = (row0 + t * C) * F
        pltpu.sync_copy(idx_hbm.at[pl.ds(ibase, CF)], idx_v)
        pltpu.async_copy(table_hbm.at[idx_v], rows_v, sem).wait()

        def row(b, carry2):
            base = b * F
            for d in range(D // L):
                acc = rows_v[base, pl.ds(d * L, L)]
                for f in range(1, F):
                    acc = acc + rows_v[base + f, pl.ds(d * L, L)]
                out_v[b, pl.ds(d * L, L)] = acc
            return carry2

        lax.fori_loop(0, C, row, 0, unroll=False)
        pltpu.sync_copy(out_v, out_hbm.at[pl.ds(row0 + t * C, C)])
        return carry

    lax.fori_loop(0, NCHUNK, chunk, 0, unroll=False)


@functools.partial(jax.jit, static_argnames=())
def _encoder_call(idx_flat, table):
    mesh = plsc.VectorSubcoreMesh(core_axis_name="c", subcore_axis_name="s")
    run = pl.kernel(
        _sc_body,
        out_type=jax.ShapeDtypeStruct((B, D), jnp.float32),
        mesh=mesh,
        scratch_types=[
            pltpu.VMEM((CF,), jnp.int32),
            pltpu.VMEM((CF, D), jnp.float32),
            pltpu.VMEM((C, D), jnp.float32),
            pltpu.SemaphoreType.DMA,
        ],
        compiler_params=pltpu.CompilerParams(use_tc_tiling_on_sc=False),
    )
    return run(idx_flat, table)


def kernel(indices, table):
    idx_flat = indices.reshape(-1).astype(jnp.int32)
    return _encoder_call(idx_flat, table)
